# Initial kernel scaffold; baseline (speedup 1.0000x reference)
import jax
import jax.numpy as jnp
from jax.experimental import pallas as pl


def kernel(voxel_xyz_t, voxel_xyz_t1, appearance_features_t, appearance_features_t1, W1, b1, W2, b2):
    raise NotImplementedError("write your pallas kernel here")


# speedup vs baseline: 31.7351x; 31.7351x over previous
import functools

import jax
import jax.numpy as jnp
from jax import lax
from jax.experimental import pallas as pl
from jax.experimental.pallas import tpu as pltpu
from jax.experimental.pallas import tpu_sc as plsc

K = 5
DIST_THRESH = 0.1
N = 8192
D = 256
E = 128
KPAD = 8

SC_NC = 2
SC_NS = 16
SC_NW = SC_NC * SC_NS
QW = N // SC_NW


def _bf16(x):
    return x.astype(jnp.bfloat16)


def _embed_body(f_ref, w1_ref, b1_ref, w2_ref, b2_ref, out_ref):
    f = f_ref[0]
    h = lax.dot_general(_bf16(f), _bf16(w1_ref[...]), (((1,), (0,)), ((), ())),
                        preferred_element_type=jnp.float32)
    h = jnp.maximum(h + b1_ref[...], 0.0)
    a = lax.dot_general(_bf16(h), _bf16(w2_ref[...]), (((1,), (0,)), ((), ())),
                        preferred_element_type=jnp.float32)
    a = a + b2_ref[...]
    nrm = jnp.sqrt(jnp.sum(a * a, axis=-1, keepdims=True))
    a = a / (nrm + 1e-8)
    out_ref[0] = _bf16(a).astype(jnp.float32)


def _embed_call(feats):
    G = feats.shape[0]
    R = 512
    def run(f, w1, b1, w2, b2):
        return pl.pallas_call(
            _embed_body,
            grid=(G, N // R),
            in_specs=[
                pl.BlockSpec((1, R, D), lambda g, i: (g, i, 0)),
                pl.BlockSpec((D, E), lambda g, i: (0, 0)),
                pl.BlockSpec((1, E), lambda g, i: (0, 0)),
                pl.BlockSpec((E, E), lambda g, i: (0, 0)),
                pl.BlockSpec((1, E), lambda g, i: (0, 0)),
            ],
            out_specs=pl.BlockSpec((1, R, E), lambda g, i: (g, i, 0)),
            out_shape=jax.ShapeDtypeStruct((G, N, E), jnp.float32),
        )(f, w1, b1, w2, b2)
    return run


_TOP5_Q = 128


def _top5_body(x_ref, yT_ref, d_ref, i_ref):
    x = x_ref[0]
    y = yT_ref[0]
    x0 = x[:, 0:1]; x1 = x[:, 1:2]; x2 = x[:, 2:3]
    sx = (x0 * x0 + x1 * x1) + x2 * x2
    y0 = y[0:1, :]; y1 = y[1:2, :]; y2 = y[2:3, :]
    sy = (y0 * y0 + y1 * y1) + y2 * y2
    cross = lax.dot_general(_bf16(x), _bf16(y), (((1,), (0,)), ((), ())),
                            preferred_element_type=jnp.float32)
    d2 = (sx - 2.0 * cross) + sy
    dist = jnp.sqrt(jnp.maximum(d2, 0.0))
    iota = lax.broadcasted_iota(jnp.int32, dist.shape, 1)
    work = dist
    ms = []
    js = []
    for _ in range(K):
        m = jnp.min(work, axis=1, keepdims=True)
        j = jnp.min(jnp.where(work == m, iota, N), axis=1, keepdims=True)
        ms.append(m)
        js.append(j)
        work = jnp.where(iota == j, jnp.inf, work)
    d5 = jnp.concatenate(ms + [ms[-1]] * (KPAD - K), axis=1)
    j5 = jnp.concatenate(js + [js[-1]] * (KPAD - K), axis=1)
    d_ref[0] = d5
    i_ref[0] = j5


def _top5_call(xyz_t, xyz_t1T):
    B = xyz_t.shape[0]
    Q = _TOP5_Q
    return pl.pallas_call(
        _top5_body,
        grid=(B, N // Q),
        in_specs=[
            pl.BlockSpec((1, Q, 3), lambda b, i: (b, i, 0)),
            pl.BlockSpec((1, 3, N), lambda b, i: (b, 0, 0)),
        ],
        out_specs=[
            pl.BlockSpec((1, Q, KPAD), lambda b, i: (b, i, 0)),
            pl.BlockSpec((1, Q, KPAD), lambda b, i: (b, i, 0)),
        ],
        out_shape=[
            jax.ShapeDtypeStruct((B, N, KPAD), jnp.float32),
            jax.ShapeDtypeStruct((B, N, KPAD), jnp.int32),
        ],
    )(xyz_t, xyz_t1T)


def _sc_rescore_body(at_hbm, at1_hbm, knni_hbm, knnd_hbm, xyzt_hbm, xyz1_hbm,
                     idx_out, conf_out, sel_out, vel_out,
                     at_v, cand_v, idxf_v, knndf_v,
                     idxk0, idxk1, idxk2, idxk3, idxk4, gidx_v,
                     sims0, sims1, sims2, sims3, sims4,
                     bi_v, conf_v, g0_v, g1_v, vel_v, sem):
    idxk = (idxk0, idxk1, idxk2, idxk3, idxk4)
    sims = (sims0, sims1, sims2, sims3, sims4)
    wid = lax.axis_index("s") * SC_NC + lax.axis_index("c")
    iota16 = lax.iota(jnp.int32, 16)
    for b in range(2):
        base = wid * QW
        row0 = b * N + base
        pltpu.sync_copy(knni_hbm.at[pl.ds(row0 * KPAD, QW * KPAD)], idxf_v)
        pltpu.sync_copy(knnd_hbm.at[pl.ds(row0 * KPAD, QW * KPAD)], knndf_v)
        pltpu.sync_copy(at_hbm.at[pl.ds(row0, QW)], at_v)

        def _build(c, carry):
            q16 = c * 16 + iota16
            for k in range(K):
                col = plsc.load_gather(idxf_v, [q16 * KPAD + k])
                idxk[k][pl.ds(c * 16, 16)] = col + b * N
            return carry
        lax.fori_loop(0, QW // 16, _build, 0)

        for k in range(K):
            pltpu.async_copy(at1_hbm.at[idxk[k]], cand_v, sem).wait()

            def _dot(q, carry, _k=k):
                acc = at_v[q, pl.ds(0, 16)] * cand_v[q, pl.ds(0, 16)]
                for c in range(1, E // 16):
                    acc = acc + (at_v[q, pl.ds(c * 16, 16)]
                                 * cand_v[q, pl.ds(c * 16, 16)])
                s = jnp.sum(acc)
                plsc.store_scatter(sims[_k], [jnp.broadcast_to(q, (16,))],
                                   jnp.broadcast_to(s, (16,)),
                                   mask=iota16 == lax.rem(q, 16))
                return carry
            lax.fori_loop(0, QW, _dot, 0)

        def _pick(c, carry):
            sl = pl.ds(c * 16, 16)
            best = sims[0][sl]
            bk = jnp.zeros((16,), jnp.int32)
            for k in range(1, K):
                s = sims[k][sl]
                gt = s > best
                best = jnp.where(gt, s, best)
                bk = jnp.where(gt, jnp.full((16,), k, jnp.int32), bk)
            flat = (c * 16 + iota16) * KPAD + bk
            bi = plsc.load_gather(idxf_v, [flat])
            bd = plsc.load_gather(knndf_v, [flat])
            conf = (0.5 * best + 0.5) * jnp.exp(-bd / DIST_THRESH)
            bi_v[sl] = bi
            conf_v[sl] = conf
            gidx_v[sl] = bi + b * N
            return carry
        lax.fori_loop(0, QW // 16, _pick, 0)

        pltpu.async_copy(xyz1_hbm.at[gidx_v], g1_v, sem).wait()
        pltpu.sync_copy(xyzt_hbm.at[pl.ds(row0, QW)], g0_v)

        def _vel(q, carry):
            vel_v[q] = g1_v[q] - g0_v[q]
            return carry
        lax.fori_loop(0, QW, _vel, 0)

        pltpu.sync_copy(bi_v, idx_out.at[pl.ds(row0, QW)])
        pltpu.sync_copy(conf_v, conf_out.at[pl.ds(row0, QW)])
        pltpu.sync_copy(g1_v, sel_out.at[pl.ds(row0, QW)])
        pltpu.sync_copy(vel_v, vel_out.at[pl.ds(row0, QW)])


def _sc_rescore(a_t, a_t1, knn_i, knn_d, xyzt_pad, xyz1_pad):
    mesh = plsc.VectorSubcoreMesh(core_axis_name="c", subcore_axis_name="s")
    fn = functools.partial(
        pl.kernel,
        out_type=(
            jax.ShapeDtypeStruct((2 * N,), jnp.int32),
            jax.ShapeDtypeStruct((2 * N,), jnp.float32),
            jax.ShapeDtypeStruct((2 * N, 16), jnp.float32),
            jax.ShapeDtypeStruct((2 * N, 16), jnp.float32),
        ),
        mesh=mesh,
        compiler_params=pltpu.CompilerParams(needs_layout_passes=False,
                                             use_tc_tiling_on_sc=False),
        scratch_types=[
            pltpu.VMEM((QW, E), jnp.float32),
            pltpu.VMEM((QW, E), jnp.float32),
            pltpu.VMEM((QW * KPAD,), jnp.int32),
            pltpu.VMEM((QW * KPAD,), jnp.float32),
            pltpu.VMEM((QW,), jnp.int32),
            pltpu.VMEM((QW,), jnp.int32),
            pltpu.VMEM((QW,), jnp.int32),
            pltpu.VMEM((QW,), jnp.int32),
            pltpu.VMEM((QW,), jnp.int32),
            pltpu.VMEM((QW,), jnp.int32),
            pltpu.VMEM((QW,), jnp.float32),
            pltpu.VMEM((QW,), jnp.float32),
            pltpu.VMEM((QW,), jnp.float32),
            pltpu.VMEM((QW,), jnp.float32),
            pltpu.VMEM((QW,), jnp.float32),
            pltpu.VMEM((QW,), jnp.int32),
            pltpu.VMEM((QW,), jnp.float32),
            pltpu.VMEM((QW, 16), jnp.float32),
            pltpu.VMEM((QW, 16), jnp.float32),
            pltpu.VMEM((QW, 16), jnp.float32),
            pltpu.SemaphoreType.DMA,
        ],
    )(_sc_rescore_body)
    return fn(a_t, a_t1, knn_i, knn_d, xyzt_pad, xyz1_pad)


def kernel(voxel_xyz_t, voxel_xyz_t1, appearance_features_t,
           appearance_features_t1, W1, b1, W2, b2):
    B = voxel_xyz_t.shape[0]

    feats = jnp.concatenate([appearance_features_t, appearance_features_t1], 0)
    emb = _embed_call(feats)(feats, W1, b1.reshape(1, E), W2, b2.reshape(1, E))
    a_t = emb[:B].reshape(B * N, E)
    a_t1 = emb[B:].reshape(B * N, E)

    xyz_t1T = jnp.transpose(voxel_xyz_t1, (0, 2, 1))
    knn_d, knn_i = _top5_call(voxel_xyz_t, xyz_t1T)

    knn_i_flat = knn_i.reshape(B * N * KPAD)
    knn_d_flat = knn_d.reshape(B * N * KPAD)
    xyzt_pad = jnp.pad(voxel_xyz_t, ((0, 0), (0, 0), (0, 13))).reshape(B * N, 16)
    xyz1_pad = jnp.pad(voxel_xyz_t1, ((0, 0), (0, 0), (0, 13))).reshape(B * N, 16)

    idx_flat, conf_flat, sel, vel = _sc_rescore(
        a_t, a_t1, knn_i_flat, knn_d_flat, xyzt_pad, xyz1_pad)

    idx_t1 = idx_flat.reshape(B, N)
    conf = conf_flat.reshape(B, N)
    xyz_m_t1 = sel.reshape(B, N, 16)[..., :3]
    velocities = vel.reshape(B, N, 16)[..., :3]

    matched_indices_t = jnp.broadcast_to(
        jnp.arange(N, dtype=jnp.int32)[None, :], (B, N))
    trajectories = jnp.stack([voxel_xyz_t, xyz_m_t1], axis=2)
    return matched_indices_t, idx_t1, conf, trajectories, velocities


# speedup vs baseline: 40.0183x; 1.2610x over previous
import functools

import jax
import jax.numpy as jnp
from jax import lax
from jax.experimental import pallas as pl
from jax.experimental.pallas import tpu as pltpu
from jax.experimental.pallas import tpu_sc as plsc

K = 5
DIST_THRESH = 0.1
N = 8192
D = 256
E = 128
KPAD = 8

SC_NC = 2
SC_NS = 16
SC_NW = SC_NC * SC_NS
QW = N // SC_NW


def _bf16(x):
    return x.astype(jnp.bfloat16)


def _embed_body(f_ref, w1_ref, b1_ref, w2_ref, b2_ref, out_ref):
    f = f_ref[0]
    h = lax.dot_general(_bf16(f), _bf16(w1_ref[...]), (((1,), (0,)), ((), ())),
                        preferred_element_type=jnp.float32)
    h = jnp.maximum(h + b1_ref[...], 0.0)
    a = lax.dot_general(_bf16(h), _bf16(w2_ref[...]), (((1,), (0,)), ((), ())),
                        preferred_element_type=jnp.float32)
    a = a + b2_ref[...]
    nrm = jnp.sqrt(jnp.sum(a * a, axis=-1, keepdims=True))
    a = a / (nrm + 1e-8)
    out_ref[0] = _bf16(a).astype(jnp.float32)


def _embed_call(feats):
    G = feats.shape[0]
    R = 512
    def run(f, w1, b1, w2, b2):
        return pl.pallas_call(
            _embed_body,
            grid=(G, N // R),
            in_specs=[
                pl.BlockSpec((1, R, D), lambda g, i: (g, i, 0)),
                pl.BlockSpec((D, E), lambda g, i: (0, 0)),
                pl.BlockSpec((1, E), lambda g, i: (0, 0)),
                pl.BlockSpec((E, E), lambda g, i: (0, 0)),
                pl.BlockSpec((1, E), lambda g, i: (0, 0)),
            ],
            out_specs=pl.BlockSpec((1, R, E), lambda g, i: (g, i, 0)),
            out_shape=jax.ShapeDtypeStruct((G, N, E), jnp.float32),
        )(f, w1, b1, w2, b2)
    return run


_TOP5_Q = 128


def _top5_body(x_ref, yT_ref, d_ref, i_ref):
    x = x_ref[0]
    y = yT_ref[0]
    x0 = x[:, 0:1]; x1 = x[:, 1:2]; x2 = x[:, 2:3]
    sx = (x0 * x0 + x1 * x1) + x2 * x2
    y0 = y[0:1, :]; y1 = y[1:2, :]; y2 = y[2:3, :]
    sy = (y0 * y0 + y1 * y1) + y2 * y2
    cross = lax.dot_general(_bf16(x), _bf16(y), (((1,), (0,)), ((), ())),
                            preferred_element_type=jnp.float32)
    d2 = (sx - 2.0 * cross) + sy
    key = jnp.maximum(d2, 0.0)
    Q = key.shape[0]
    NB = N // 128
    lane = lax.broadcasted_iota(jnp.int32, (Q, 128), 1)
    vals = [key[:, c * 128:(c + 1) * 128] for c in range(NB)]
    lt0 = [vals[2 * c + 1] < vals[2 * c] for c in range(NB // 2)]
    idxs = [jnp.where(lt0[c], 2 * c + 1, 2 * c) for c in range(NB // 2)]
    vals = [jnp.where(lt0[c], vals[2 * c + 1], vals[2 * c])
            for c in range(NB // 2)]
    while len(vals) > 1:
        nv, ni = [], []
        for a in range(0, len(vals), 2):
            lt = vals[a + 1] < vals[a]
            nv.append(jnp.where(lt, vals[a + 1], vals[a]))
            ni.append(jnp.where(lt, idxs[a + 1], idxs[a]))
        vals, idxs = nv, ni
    M = vals[0]
    J = idxs[0] * 128 + lane
    clss = []
    for _ in range(K):
        m = jnp.min(M, axis=1, keepdims=True)
        jm = jnp.min(jnp.where(M == m, J, N), axis=1, keepdims=True)
        clss.append(jnp.bitwise_and(jm, 127))
        M = jnp.where(J == jm, jnp.inf, M)
    lanemod8 = jnp.bitwise_and(lane, 7)
    cls128 = jnp.broadcast_to(clss[0], (Q, 128))
    for k in range(1, K):
        cls128 = jnp.where(lanemod8 == k, jnp.broadcast_to(clss[k], (Q, 128)),
                           cls128)
    padmask = lanemod8 >= K
    groupbase = jnp.bitwise_and(lane, 127 - 7) * 16
    NW_B = NB // 16
    Ws = []
    jcs = []
    for w in range(NW_B):
        W = None
        for g in range(16):
            c = w * 16 + g
            gat = jnp.take_along_axis(key[:, c * 128:(c + 1) * 128], cls128,
                                      axis=1)
            if W is None:
                W = gat
            else:
                W = jnp.where(jnp.bitwise_and(lane, 127 - 7) == g * 8, gat, W)
        Ws.append(jnp.where(padmask, jnp.inf, W))
        jcs.append(cls128 + (groupbase + w * 2048))
    work = jnp.concatenate(Ws, axis=1)
    jc = jnp.concatenate(jcs, axis=1)
    ms = []
    js = []
    for _ in range(K):
        m = jnp.min(work, axis=1, keepdims=True)
        j = jnp.min(jnp.where(work == m, jc, N), axis=1, keepdims=True)
        ms.append(m)
        js.append(j)
        work = jnp.where(jc == j, jnp.inf, work)
    d5 = jnp.concatenate(ms + [ms[-1]] * (KPAD - K), axis=1)
    j5 = jnp.concatenate(js + [js[-1]] * (KPAD - K), axis=1)
    d_ref[0] = jnp.sqrt(d5)
    i_ref[0] = j5


def _top5_call(xyz_t, xyz_t1T):
    B = xyz_t.shape[0]
    Q = _TOP5_Q
    return pl.pallas_call(
        _top5_body,
        grid=(B, N // Q),
        in_specs=[
            pl.BlockSpec((1, Q, 3), lambda b, i: (b, i, 0)),
            pl.BlockSpec((1, 3, N), lambda b, i: (b, 0, 0)),
        ],
        out_specs=[
            pl.BlockSpec((1, Q, KPAD), lambda b, i: (b, i, 0)),
            pl.BlockSpec((1, Q, KPAD), lambda b, i: (b, i, 0)),
        ],
        out_shape=[
            jax.ShapeDtypeStruct((B, N, KPAD), jnp.float32),
            jax.ShapeDtypeStruct((B, N, KPAD), jnp.int32),
        ],
    )(xyz_t, xyz_t1T)


def _sc_rescore_body(at_hbm, at1_hbm, knni_hbm, knnd_hbm, xyzt_hbm, xyz1_hbm,
                     idx_out, conf_out, sel_out, vel_out,
                     at_v, cand_v, idxf_v, knndf_v,
                     idxk0, idxk1, idxk2, idxk3, idxk4, gidx_v,
                     sims0, sims1, sims2, sims3, sims4,
                     bi_v, conf_v, g0_v, g1_v, vel_v, sem):
    idxk = (idxk0, idxk1, idxk2, idxk3, idxk4)
    sims = (sims0, sims1, sims2, sims3, sims4)
    wid = lax.axis_index("s") * SC_NC + lax.axis_index("c")
    iota16 = lax.iota(jnp.int32, 16)
    for b in range(2):
        base = wid * QW
        row0 = b * N + base
        pltpu.sync_copy(knni_hbm.at[pl.ds(row0 * KPAD, QW * KPAD)], idxf_v)
        pltpu.sync_copy(knnd_hbm.at[pl.ds(row0 * KPAD, QW * KPAD)], knndf_v)
        pltpu.sync_copy(at_hbm.at[pl.ds(row0, QW)], at_v)

        def _build(c, carry):
            q16 = c * 16 + iota16
            for k in range(K):
                col = plsc.load_gather(idxf_v, [q16 * KPAD + k])
                idxk[k][pl.ds(c * 16, 16)] = col + b * N
            return carry
        lax.fori_loop(0, QW // 16, _build, 0)

        for k in range(K):
            pltpu.async_copy(at1_hbm.at[idxk[k]], cand_v, sem).wait()

            def _dot(q, carry, _k=k):
                acc = at_v[q, pl.ds(0, 16)] * cand_v[q, pl.ds(0, 16)]
                for c in range(1, E // 16):
                    acc = acc + (at_v[q, pl.ds(c * 16, 16)]
                                 * cand_v[q, pl.ds(c * 16, 16)])
                s = jnp.sum(acc)
                plsc.store_scatter(sims[_k], [jnp.broadcast_to(q, (16,))],
                                   jnp.broadcast_to(s, (16,)),
                                   mask=iota16 == lax.rem(q, 16))
                return carry
            lax.fori_loop(0, QW, _dot, 0)

        def _pick(c, carry):
            sl = pl.ds(c * 16, 16)
            best = sims[0][sl]
            bk = jnp.zeros((16,), jnp.int32)
            for k in range(1, K):
                s = sims[k][sl]
                gt = s > best
                best = jnp.where(gt, s, best)
                bk = jnp.where(gt, jnp.full((16,), k, jnp.int32), bk)
            flat = (c * 16 + iota16) * KPAD + bk
            bi = plsc.load_gather(idxf_v, [flat])
            bd = plsc.load_gather(knndf_v, [flat])
            conf = (0.5 * best + 0.5) * jnp.exp(-bd / DIST_THRESH)
            bi_v[sl] = bi
            conf_v[sl] = conf
            gidx_v[sl] = bi + b * N
            return carry
        lax.fori_loop(0, QW // 16, _pick, 0)

        pltpu.async_copy(xyz1_hbm.at[gidx_v], g1_v, sem).wait()
        pltpu.sync_copy(xyzt_hbm.at[pl.ds(row0, QW)], g0_v)

        def _vel(q, carry):
            vel_v[q] = g1_v[q] - g0_v[q]
            return carry
        lax.fori_loop(0, QW, _vel, 0)

        pltpu.sync_copy(bi_v, idx_out.at[pl.ds(row0, QW)])
        pltpu.sync_copy(conf_v, conf_out.at[pl.ds(row0, QW)])
        pltpu.sync_copy(g1_v, sel_out.at[pl.ds(row0, QW)])
        pltpu.sync_copy(vel_v, vel_out.at[pl.ds(row0, QW)])


def _sc_rescore(a_t, a_t1, knn_i, knn_d, xyzt_pad, xyz1_pad):
    mesh = plsc.VectorSubcoreMesh(core_axis_name="c", subcore_axis_name="s")
    fn = functools.partial(
        pl.kernel,
        out_type=(
            jax.ShapeDtypeStruct((2 * N,), jnp.int32),
            jax.ShapeDtypeStruct((2 * N,), jnp.float32),
            jax.ShapeDtypeStruct((2 * N, 16), jnp.float32),
            jax.ShapeDtypeStruct((2 * N, 16), jnp.float32),
        ),
        mesh=mesh,
        compiler_params=pltpu.CompilerParams(needs_layout_passes=False,
                                             use_tc_tiling_on_sc=False),
        scratch_types=[
            pltpu.VMEM((QW, E), jnp.float32),
            pltpu.VMEM((QW, E), jnp.float32),
            pltpu.VMEM((QW * KPAD,), jnp.int32),
            pltpu.VMEM((QW * KPAD,), jnp.float32),
            pltpu.VMEM((QW,), jnp.int32),
            pltpu.VMEM((QW,), jnp.int32),
            pltpu.VMEM((QW,), jnp.int32),
            pltpu.VMEM((QW,), jnp.int32),
            pltpu.VMEM((QW,), jnp.int32),
            pltpu.VMEM((QW,), jnp.int32),
            pltpu.VMEM((QW,), jnp.float32),
            pltpu.VMEM((QW,), jnp.float32),
            pltpu.VMEM((QW,), jnp.float32),
            pltpu.VMEM((QW,), jnp.float32),
            pltpu.VMEM((QW,), jnp.float32),
            pltpu.VMEM((QW,), jnp.int32),
            pltpu.VMEM((QW,), jnp.float32),
            pltpu.VMEM((QW, 16), jnp.float32),
            pltpu.VMEM((QW, 16), jnp.float32),
            pltpu.VMEM((QW, 16), jnp.float32),
            pltpu.SemaphoreType.DMA,
        ],
    )(_sc_rescore_body)
    return fn(a_t, a_t1, knn_i, knn_d, xyzt_pad, xyz1_pad)


def kernel(voxel_xyz_t, voxel_xyz_t1, appearance_features_t,
           appearance_features_t1, W1, b1, W2, b2):
    B = voxel_xyz_t.shape[0]

    feats = jnp.concatenate([appearance_features_t, appearance_features_t1], 0)
    emb = _embed_call(feats)(feats, W1, b1.reshape(1, E), W2, b2.reshape(1, E))
    a_t = emb[:B].reshape(B * N, E)
    a_t1 = emb[B:].reshape(B * N, E)

    xyz_t1T = jnp.transpose(voxel_xyz_t1, (0, 2, 1))
    knn_d, knn_i = _top5_call(voxel_xyz_t, xyz_t1T)

    knn_i_flat = knn_i.reshape(B * N * KPAD)
    knn_d_flat = knn_d.reshape(B * N * KPAD)
    xyzt_pad = jnp.pad(voxel_xyz_t, ((0, 0), (0, 0), (0, 13))).reshape(B * N, 16)
    xyz1_pad = jnp.pad(voxel_xyz_t1, ((0, 0), (0, 0), (0, 13))).reshape(B * N, 16)

    idx_flat, conf_flat, sel, vel = _sc_rescore(
        a_t, a_t1, knn_i_flat, knn_d_flat, xyzt_pad, xyz1_pad)

    idx_t1 = idx_flat.reshape(B, N)
    conf = conf_flat.reshape(B, N)
    xyz_m_t1 = sel.reshape(B, N, 16)[..., :3]
    velocities = vel.reshape(B, N, 16)[..., :3]

    matched_indices_t = jnp.broadcast_to(
        jnp.arange(N, dtype=jnp.int32)[None, :], (B, N))
    trajectories = jnp.stack([voxel_xyz_t, xyz_m_t1], axis=2)
    return matched_indices_t, idx_t1, conf, trajectories, velocities
